# baseline (speedup 1.0000x reference)
import functools

import jax
import jax.numpy as jnp
from jax import lax
from jax.experimental import pallas as pl
from jax.experimental.pallas import tpu as pltpu
from jax.experimental.pallas import tpu_sc as plsc

TOP_K = 2
NUM_EXPERTS = 8
D_MODEL = 1024
TOKENS = 4096
D32 = D_MODEL // 2

RBLK = 256
NBLK = 40
PADDED = NBLK * RBLK

CHUNK = 128
NCHUNK = TOKENS // CHUNK

NWORK = 32
ROWS_PER_W = PADDED // NWORK
TOK_PER_W = TOKENS // NWORK


def _tri_left(n, strict):
    r = lax.broadcasted_iota(jnp.int32, (n, n), 0)
    c = lax.broadcasted_iota(jnp.int32, (n, n), 1)
    return jnp.where((r > c) if strict else (r >= c), 1.0, 0.0)


def _tri_right(n):
    r = lax.broadcasted_iota(jnp.int32, (n, n), 0)
    c = lax.broadcasted_iota(jnp.int32, (n, n), 1)
    return jnp.where(r <= c, 1.0, 0.0)


def _router_kernel(x_ref, wg_ref, bg_ref, probs_ref, pos_ref,
                   bexp_ref, bias_ref):
    x = x_ref[...]
    scores = jnp.dot(x, wg_ref[...], preferred_element_type=jnp.float32)
    scores = scores + bg_ref[...]
    idx = lax.broadcasted_iota(jnp.int32, scores.shape, 1)
    m1 = jnp.max(scores, axis=1, keepdims=True)
    i1 = jnp.min(jnp.where(scores == m1, idx, NUM_EXPERTS), axis=1,
                 keepdims=True)
    masked = jnp.where(idx == i1, -jnp.inf, scores)
    m2 = jnp.max(masked, axis=1, keepdims=True)
    i2 = jnp.min(jnp.where(masked == m2, idx, NUM_EXPERTS), axis=1,
                 keepdims=True)
    e2 = jnp.exp(m2 - m1)
    denom = 1.0 + e2
    p0 = 1.0 / denom
    p1 = e2 / denom
    probs_ref[...] = jnp.concatenate([p0, p1], axis=1)

    oh0 = jnp.where(idx == i1, 1.0, 0.0)
    oh1 = jnp.where(idx == i2, 1.0, 0.0)
    oh = jnp.concatenate([oh0, oh1], axis=1)

    t_in = _tri_left(CHUNK, strict=False)
    incl_chunks = []
    last_rows = []
    for c in range(NCHUNK):
        blk = lax.slice(oh, (c * CHUNK, 0), ((c + 1) * CHUNK, 2 * NUM_EXPERTS))
        inc = jnp.dot(t_in, blk, preferred_element_type=jnp.float32)
        incl_chunks.append(inc)
        last_rows.append(lax.slice(inc, (CHUNK - 1, 0),
                                   (CHUNK, 2 * NUM_EXPERTS)))
    p_sums = jnp.concatenate(last_rows, axis=0)
    t_ex = _tri_left(NCHUNK, strict=True)
    chunk_prefix = jnp.dot(t_ex, p_sums,
                           preferred_element_type=jnp.float32)
    full = jnp.concatenate(
        [incl_chunks[c] + lax.slice(chunk_prefix, (c, 0),
                                    (c + 1, 2 * NUM_EXPERTS))
         for c in range(NCHUNK)], axis=0)
    c0 = full[:, :NUM_EXPERTS]
    c1 = full[:, NUM_EXPERTS:]
    totals = jnp.sum(p_sums, axis=0, keepdims=True)
    count0 = totals[:, :NUM_EXPERTS]
    count1 = totals[:, NUM_EXPERTS:]
    counts = count0 + count1

    nb = jnp.floor((counts + (RBLK - 1)) * (1.0 / RBLK))
    incl_nb = jnp.dot(nb, _tri_right(NUM_EXPERTS),
                      preferred_element_type=jnp.float32)
    start_rows = (incl_nb - nb) * RBLK

    rank0 = jnp.sum(oh0 * c0, axis=1, keepdims=True) - 1.0
    rank1 = jnp.sum(oh1 * (count0 + c1), axis=1, keepdims=True) - 1.0
    base0 = jnp.sum(oh0 * start_rows, axis=1, keepdims=True)
    base1 = jnp.sum(oh1 * start_rows, axis=1, keepdims=True)
    pos0 = (base0 + rank0).astype(jnp.int32)
    pos1 = (base1 + rank1).astype(jnp.int32)
    pos_ref[...] = jnp.concatenate([pos0, pos1], axis=1)

    bidx = lax.broadcasted_iota(jnp.int32, (1, 128), 1).astype(jnp.float32)
    bexp = jnp.zeros((1, 128), jnp.float32)
    for e in range(NUM_EXPERTS):
        ends_e = lax.slice(incl_nb, (0, e), (1, e + 1))
        bexp = bexp + jnp.where(bidx >= ends_e, 1.0, 0.0)
    bexp_ref[...] = jnp.minimum(bexp, float(NUM_EXPERTS - 1)).astype(jnp.int32)

    bias_ref[...] = oh0 * p0 + oh1 * p1


def _router(inputs, Wg, bg):
    return pl.pallas_call(
        _router_kernel,
        grid=(1,),
        in_specs=[
            pl.BlockSpec((TOKENS, D_MODEL), lambda i: (0, 0)),
            pl.BlockSpec((D_MODEL, NUM_EXPERTS), lambda i: (0, 0)),
            pl.BlockSpec((1, NUM_EXPERTS), lambda i: (0, 0)),
        ],
        out_specs=[
            pl.BlockSpec((TOKENS, TOP_K), lambda i: (0, 0)),
            pl.BlockSpec((TOKENS, TOP_K), lambda i: (0, 0)),
            pl.BlockSpec((1, 128), lambda i: (0, 0)),
            pl.BlockSpec((TOKENS, NUM_EXPERTS), lambda i: (0, 0)),
        ],
        out_shape=[
            jax.ShapeDtypeStruct((TOKENS, TOP_K), jnp.float32),
            jax.ShapeDtypeStruct((TOKENS, TOP_K), jnp.int32),
            jax.ShapeDtypeStruct((1, 128), jnp.int32),
            jax.ShapeDtypeStruct((TOKENS, NUM_EXPERTS), jnp.float32),
        ],
    )(inputs, Wg, bg.reshape(1, NUM_EXPERTS))


def _bias_kernel(h_ref, be_ref, out_ref):
    out_ref[...] = jnp.dot(h_ref[...], be_ref[...],
                           preferred_element_type=jnp.float32,
                           precision=lax.Precision.HIGHEST)


def _bias_comb(hcomb, be):
    return pl.pallas_call(
        _bias_kernel,
        grid=(8,),
        in_specs=[
            pl.BlockSpec((TOKENS // 8, NUM_EXPERTS), lambda i: (i, 0)),
            pl.BlockSpec((NUM_EXPERTS, D_MODEL), lambda i: (0, 0)),
        ],
        out_specs=pl.BlockSpec((TOKENS // 8, D_MODEL), lambda i: (i, 0)),
        out_shape=jax.ShapeDtypeStruct((TOKENS, D_MODEL), jnp.float32),
    )(hcomb, be)


GCHUNK = 64


def _dispatch_body(pos_hbm, xb_hbm, xs_hbm, posbuf, sortids, rowbuf, sem):
    wid = lax.axis_index("s") * 2 + lax.axis_index("c")
    pltpu.sync_copy(pos_hbm, posbuf)

    def zero_body(i, _):
        sortids[pl.ds(i * 16, 16)] = jnp.zeros((16,), jnp.int32)
        return 0

    lax.fori_loop(0, PADDED // 16, zero_body, 0)

    def scat_body(c, _):
        av = c * 16 + lax.iota(jnp.int32, 16)
        tok = lax.shift_right_arithmetic(av, 1)
        pv = posbuf[pl.ds(c * 16, 16)]
        plsc.store_scatter(sortids, [pv], tok)
        return 0

    lax.fori_loop(0, (TOKENS * TOP_K) // 16, scat_body, 0)

    base = wid * ROWS_PER_W
    for j in range(ROWS_PER_W // GCHUNK):
        idx_sl = sortids.at[pl.ds(base + j * GCHUNK, GCHUNK)]
        pltpu.async_copy(xb_hbm.at[idx_sl], rowbuf, sem).wait()
        pltpu.sync_copy(rowbuf, xs_hbm.at[pl.ds(base + j * GCHUNK, GCHUNK)])


def _dispatch(posflat, xb32):
    mesh = plsc.VectorSubcoreMesh(core_axis_name="c", subcore_axis_name="s")
    return pl.kernel(
        _dispatch_body,
        mesh=mesh,
        out_type=jax.ShapeDtypeStruct((PADDED, D32), jnp.int32),
        compiler_params=pltpu.CompilerParams(needs_layout_passes=False),
        scratch_types=[
            pltpu.VMEM((TOKENS * TOP_K,), jnp.int32),
            pltpu.VMEM((PADDED,), jnp.int32),
            pltpu.VMEM((GCHUNK, D32), jnp.int32),
            pltpu.SemaphoreType.DMA,
        ],
    )(posflat, xb32)


def _gmm_kernel(bexp_ref, xs_ref, we_ref, ys_ref):
    del bexp_ref
    ys_ref[...] = jnp.dot(xs_ref[...], we_ref[0],
                          preferred_element_type=jnp.float32)


def _grouped_matmul(block_expert, xs_bf16, We_bf16):
    grid_spec = pltpu.PrefetchScalarGridSpec(
        num_scalar_prefetch=1,
        grid=(NBLK,),
        in_specs=[
            pl.BlockSpec((RBLK, D_MODEL), lambda i, be: (i, 0)),
            pl.BlockSpec((1, D_MODEL, D_MODEL), lambda i, be: (be[i], 0, 0)),
        ],
        out_specs=pl.BlockSpec((RBLK, D_MODEL), lambda i, be: (i, 0)),
    )
    return pl.pallas_call(
        _gmm_kernel,
        grid_spec=grid_spec,
        out_shape=jax.ShapeDtypeStruct((PADDED, D_MODEL), jnp.float32),
    )(block_expert, xs_bf16, We_bf16)


CTOK = 16


def _combine_body(pos_hbm, gates_hbm, bias_hbm, ys_hbm, out_hbm,
                  posbuf, gbuf, rowbuf, biasbuf, outbuf, sem):
    wid = lax.axis_index("s") * 2 + lax.axis_index("c")
    tbase = wid * TOK_PER_W
    pltpu.sync_copy(pos_hbm.at[pl.ds(TOP_K * tbase, TOP_K * TOK_PER_W)],
                    posbuf)
    pltpu.sync_copy(gates_hbm.at[pl.ds(TOP_K * tbase, TOP_K * TOK_PER_W)],
                    gbuf)
    for j in range(TOK_PER_W // CTOK):
        t0 = tbase + j * CTOK
        idx_sl = posbuf.at[pl.ds(j * 2 * CTOK, 2 * CTOK)]
        pltpu.async_copy(ys_hbm.at[idx_sl], rowbuf, sem).wait()
        pltpu.sync_copy(bias_hbm.at[pl.ds(t0, CTOK)], biasbuf)

        def tok_body(i, _):
            gi = j * 2 * CTOK + 2 * i
            g0 = plsc.load_gather(gbuf, [jnp.full((16,), gi, jnp.int32)])
            g1 = plsc.load_gather(gbuf, [jnp.full((16,), gi + 1, jnp.int32)])
            for c in range(D_MODEL // 16):
                sl = pl.ds(c * 16, 16)
                outbuf[i, sl] = (g0 * rowbuf[2 * i, sl] +
                                 g1 * rowbuf[2 * i + 1, sl] + biasbuf[i, sl])
            return 0

        lax.fori_loop(0, CTOK, tok_body, 0)
        pltpu.sync_copy(outbuf, out_hbm.at[pl.ds(t0, CTOK)])


def _combine(posflat, gates, bias_comb, ys):
    mesh = plsc.VectorSubcoreMesh(core_axis_name="c", subcore_axis_name="s")
    return pl.kernel(
        _combine_body,
        mesh=mesh,
        out_type=jax.ShapeDtypeStruct((TOKENS, D_MODEL), jnp.float32),
        compiler_params=pltpu.CompilerParams(needs_layout_passes=False),
        scratch_types=[
            pltpu.VMEM((TOP_K * TOK_PER_W,), jnp.int32),
            pltpu.VMEM((TOP_K * TOK_PER_W,), jnp.float32),
            pltpu.VMEM((2 * CTOK, D_MODEL), jnp.float32),
            pltpu.VMEM((CTOK, D_MODEL), jnp.float32),
            pltpu.VMEM((CTOK, D_MODEL), jnp.float32),
            pltpu.SemaphoreType.DMA,
        ],
    )(posflat, gates, bias_comb, ys)


def kernel(inputs, Wg, bg, We, be):
    xb = inputs.astype(jnp.bfloat16)
    xb32 = lax.bitcast_convert_type(
        xb.reshape(TOKENS, D32, 2), jnp.int32)

    probs, posflat2, bexp_row, hcomb = _router(inputs, Wg, bg)
    bias_comb = _bias_comb(hcomb, be)
    posflat = posflat2.reshape(TOKENS * TOP_K)
    gates = probs.reshape(TOKENS * TOP_K)
    block_expert = bexp_row[0, :NBLK]

    xs32 = _dispatch(posflat, xb32)
    xs_bf16 = lax.bitcast_convert_type(xs32, jnp.bfloat16).reshape(
        PADDED, D_MODEL)
    ys = _grouped_matmul(block_expert, xs_bf16, We.astype(jnp.bfloat16))
    out = _combine(posflat, gates, bias_comb, ys)
    return (out, probs)


# speedup vs baseline: 1.0369x; 1.0369x over previous
import functools

import jax
import jax.numpy as jnp
from jax import lax
from jax.experimental import pallas as pl
from jax.experimental.pallas import tpu as pltpu
from jax.experimental.pallas import tpu_sc as plsc

TOP_K = 2
NUM_EXPERTS = 8
D_MODEL = 1024
TOKENS = 4096
D32 = D_MODEL // 2

RBLK = 256
NBLK = 40
PADDED = NBLK * RBLK

CHUNK = 128
NCHUNK = TOKENS // CHUNK

NWORK = 32
ROWS_PER_W = PADDED // NWORK
TOK_PER_W = TOKENS // NWORK


def _tri_left(n, strict):
    r = lax.broadcasted_iota(jnp.int32, (n, n), 0)
    c = lax.broadcasted_iota(jnp.int32, (n, n), 1)
    return jnp.where((r > c) if strict else (r >= c), 1.0, 0.0)


def _tri_right(n):
    r = lax.broadcasted_iota(jnp.int32, (n, n), 0)
    c = lax.broadcasted_iota(jnp.int32, (n, n), 1)
    return jnp.where(r <= c, 1.0, 0.0)


def _router_kernel(x_ref, wg_ref, bg_ref, probs_ref, pos_ref,
                   bexp_ref, bias_ref):
    x = x_ref[...]
    scores = jnp.dot(x, wg_ref[...], preferred_element_type=jnp.float32)
    scores = scores + bg_ref[...]
    idx = lax.broadcasted_iota(jnp.int32, scores.shape, 1)
    m1 = jnp.max(scores, axis=1, keepdims=True)
    i1 = jnp.min(jnp.where(scores == m1, idx, NUM_EXPERTS), axis=1,
                 keepdims=True)
    masked = jnp.where(idx == i1, -jnp.inf, scores)
    m2 = jnp.max(masked, axis=1, keepdims=True)
    i2 = jnp.min(jnp.where(masked == m2, idx, NUM_EXPERTS), axis=1,
                 keepdims=True)
    e2 = jnp.exp(m2 - m1)
    denom = 1.0 + e2
    p0 = 1.0 / denom
    p1 = e2 / denom
    probs_ref[...] = jnp.concatenate([p0, p1], axis=1)

    oh0 = jnp.where(idx == i1, 1.0, 0.0)
    oh1 = jnp.where(idx == i2, 1.0, 0.0)
    oh = jnp.concatenate([oh0, oh1], axis=1)

    t_in = _tri_left(CHUNK, strict=False)
    incl_chunks = []
    last_rows = []
    for c in range(NCHUNK):
        blk = lax.slice(oh, (c * CHUNK, 0), ((c + 1) * CHUNK, 2 * NUM_EXPERTS))
        inc = jnp.dot(t_in, blk, preferred_element_type=jnp.float32)
        incl_chunks.append(inc)
        last_rows.append(lax.slice(inc, (CHUNK - 1, 0),
                                   (CHUNK, 2 * NUM_EXPERTS)))
    p_sums = jnp.concatenate(last_rows, axis=0)
    t_ex = _tri_left(NCHUNK, strict=True)
    chunk_prefix = jnp.dot(t_ex, p_sums,
                           preferred_element_type=jnp.float32)
    full = jnp.concatenate(
        [incl_chunks[c] + lax.slice(chunk_prefix, (c, 0),
                                    (c + 1, 2 * NUM_EXPERTS))
         for c in range(NCHUNK)], axis=0)
    c0 = full[:, :NUM_EXPERTS]
    c1 = full[:, NUM_EXPERTS:]
    totals = jnp.sum(p_sums, axis=0, keepdims=True)
    count0 = totals[:, :NUM_EXPERTS]
    count1 = totals[:, NUM_EXPERTS:]
    counts = count0 + count1

    nb = jnp.floor((counts + (RBLK - 1)) * (1.0 / RBLK))
    incl_nb = jnp.dot(nb, _tri_right(NUM_EXPERTS),
                      preferred_element_type=jnp.float32)
    start_rows = (incl_nb - nb) * RBLK

    rank0 = jnp.sum(oh0 * c0, axis=1, keepdims=True) - 1.0
    rank1 = jnp.sum(oh1 * (count0 + c1), axis=1, keepdims=True) - 1.0
    base0 = jnp.sum(oh0 * start_rows, axis=1, keepdims=True)
    base1 = jnp.sum(oh1 * start_rows, axis=1, keepdims=True)
    pos0 = (base0 + rank0).astype(jnp.int32)
    pos1 = (base1 + rank1).astype(jnp.int32)
    pos_ref[...] = jnp.concatenate([pos0, pos1], axis=1)

    bidx = lax.broadcasted_iota(jnp.int32, (1, 128), 1).astype(jnp.float32)
    bexp = jnp.zeros((1, 128), jnp.float32)
    for e in range(NUM_EXPERTS):
        ends_e = lax.slice(incl_nb, (0, e), (1, e + 1))
        bexp = bexp + jnp.where(bidx >= ends_e, 1.0, 0.0)
    bexp_ref[...] = jnp.minimum(bexp, float(NUM_EXPERTS - 1)).astype(jnp.int32)

    bias_ref[...] = oh0 * p0 + oh1 * p1


def _router(inputs, Wg, bg):
    return pl.pallas_call(
        _router_kernel,
        grid=(1,),
        in_specs=[
            pl.BlockSpec((TOKENS, D_MODEL), lambda i: (0, 0)),
            pl.BlockSpec((D_MODEL, NUM_EXPERTS), lambda i: (0, 0)),
            pl.BlockSpec((1, NUM_EXPERTS), lambda i: (0, 0)),
        ],
        out_specs=[
            pl.BlockSpec((TOKENS, TOP_K), lambda i: (0, 0)),
            pl.BlockSpec((TOKENS, TOP_K), lambda i: (0, 0)),
            pl.BlockSpec((1, 128), lambda i: (0, 0)),
            pl.BlockSpec((TOKENS, NUM_EXPERTS), lambda i: (0, 0)),
        ],
        out_shape=[
            jax.ShapeDtypeStruct((TOKENS, TOP_K), jnp.float32),
            jax.ShapeDtypeStruct((TOKENS, TOP_K), jnp.int32),
            jax.ShapeDtypeStruct((1, 128), jnp.int32),
            jax.ShapeDtypeStruct((TOKENS, NUM_EXPERTS), jnp.float32),
        ],
    )(inputs, Wg, bg.reshape(1, NUM_EXPERTS))


def _bias_kernel(h_ref, be_ref, out_ref):
    out_ref[...] = jnp.dot(h_ref[...], be_ref[...],
                           preferred_element_type=jnp.float32,
                           precision=lax.Precision.HIGHEST)


def _bias_comb(hcomb, be):
    return pl.pallas_call(
        _bias_kernel,
        grid=(8,),
        in_specs=[
            pl.BlockSpec((TOKENS // 8, NUM_EXPERTS), lambda i: (i, 0)),
            pl.BlockSpec((NUM_EXPERTS, D_MODEL), lambda i: (0, 0)),
        ],
        out_specs=pl.BlockSpec((TOKENS // 8, D_MODEL), lambda i: (i, 0)),
        out_shape=jax.ShapeDtypeStruct((TOKENS, D_MODEL), jnp.float32),
    )(hcomb, be)


GCHUNK = 64


def _dispatch_body(pos_hbm, xb_hbm, xs_hbm, posbuf, sortids, rowbuf0,
                   rowbuf1, gsem, wsem):
    wid = lax.axis_index("s") * 2 + lax.axis_index("c")
    pltpu.sync_copy(pos_hbm, posbuf)

    def zero_body(i, _):
        sortids[pl.ds(i * 16, 16)] = jnp.zeros((16,), jnp.int32)
        return 0

    lax.fori_loop(0, PADDED // 16, zero_body, 0)

    def scat_body(c, _):
        av = c * 16 + lax.iota(jnp.int32, 16)
        tok = lax.shift_right_arithmetic(av, 1)
        pv = posbuf[pl.ds(c * 16, 16)]
        plsc.store_scatter(sortids, [pv], tok)
        return 0

    lax.fori_loop(0, (TOKENS * TOP_K) // 16, scat_body, 0)

    base = wid * ROWS_PER_W
    nch = ROWS_PER_W // GCHUNK
    bufs = [rowbuf0, rowbuf1]

    def gather(j):
        idx_sl = sortids.at[pl.ds(base + j * GCHUNK, GCHUNK)]
        return pltpu.async_copy(xb_hbm.at[idx_sl], bufs[j % 2], gsem)

    def write(j):
        return pltpu.async_copy(
            bufs[j % 2], xs_hbm.at[pl.ds(base + j * GCHUNK, GCHUNK)], wsem)

    gh = [None] * nch
    wh = [None] * nch
    gh[0] = gather(0)
    for j in range(nch):
        if j + 1 < nch:
            if j >= 1:
                wh[j - 1].wait()
            gh[j + 1] = gather(j + 1)
        gh[j].wait()
        wh[j] = write(j)
    wh[nch - 2].wait()
    wh[nch - 1].wait()


def _dispatch(posflat, xb32):
    mesh = plsc.VectorSubcoreMesh(core_axis_name="c", subcore_axis_name="s")
    return pl.kernel(
        _dispatch_body,
        mesh=mesh,
        out_type=jax.ShapeDtypeStruct((PADDED, D32), jnp.int32),
        compiler_params=pltpu.CompilerParams(needs_layout_passes=False),
        scratch_types=[
            pltpu.VMEM((TOKENS * TOP_K,), jnp.int32),
            pltpu.VMEM((PADDED,), jnp.int32),
            pltpu.VMEM((GCHUNK, D32), jnp.int32),
            pltpu.VMEM((GCHUNK, D32), jnp.int32),
            pltpu.SemaphoreType.DMA,
            pltpu.SemaphoreType.DMA,
        ],
    )(posflat, xb32)


def _gmm_kernel(bexp_ref, xs_ref, we_ref, ys_ref):
    del bexp_ref
    ys_ref[...] = jnp.dot(xs_ref[...], we_ref[0],
                          preferred_element_type=jnp.float32)


def _grouped_matmul(block_expert, xs_bf16, We_bf16):
    grid_spec = pltpu.PrefetchScalarGridSpec(
        num_scalar_prefetch=1,
        grid=(NBLK,),
        in_specs=[
            pl.BlockSpec((RBLK, D_MODEL), lambda i, be: (i, 0)),
            pl.BlockSpec((1, D_MODEL, D_MODEL), lambda i, be: (be[i], 0, 0)),
        ],
        out_specs=pl.BlockSpec((RBLK, D_MODEL), lambda i, be: (i, 0)),
    )
    return pl.pallas_call(
        _gmm_kernel,
        grid_spec=grid_spec,
        out_shape=jax.ShapeDtypeStruct((PADDED, D_MODEL), jnp.float32),
    )(block_expert, xs_bf16, We_bf16)


CTOK = 8


def _combine_body(pos_hbm, gates_hbm, bias_hbm, ys_hbm, out_hbm,
                  posbuf, gbuf, row0, row1, bias0, bias1, out0, out1,
                  gsem, bsem, osem):
    wid = lax.axis_index("s") * 2 + lax.axis_index("c")
    tbase = wid * TOK_PER_W
    nch = TOK_PER_W // CTOK
    rows = [row0, row1]
    biases = [bias0, bias1]
    outs = [out0, out1]
    pltpu.sync_copy(pos_hbm.at[pl.ds(TOP_K * tbase, TOP_K * TOK_PER_W)],
                    posbuf)
    pltpu.sync_copy(gates_hbm.at[pl.ds(TOP_K * tbase, TOP_K * TOK_PER_W)],
                    gbuf)

    def fetch(j):
        idx_sl = posbuf.at[pl.ds(j * 2 * CTOK, 2 * CTOK)]
        gh = pltpu.async_copy(ys_hbm.at[idx_sl], rows[j % 2], gsem)
        bh = pltpu.async_copy(bias_hbm.at[pl.ds(tbase + j * CTOK, CTOK)],
                              biases[j % 2], bsem)
        return gh, bh

    pend = [None] * nch
    wh = [None] * nch
    pend[0] = fetch(0)
    for j in range(nch):
        if j + 1 < nch:
            if j >= 1:
                wh[j - 1].wait()
            pend[j + 1] = fetch(j + 1)
        pend[j][0].wait()
        pend[j][1].wait()
        rowbuf = rows[j % 2]
        biasbuf = biases[j % 2]
        outbuf = outs[j % 2]

        def tok_body(i, _):
            gi = j * 2 * CTOK + 2 * i
            g0 = plsc.load_gather(gbuf, [jnp.full((16,), gi, jnp.int32)])
            g1 = plsc.load_gather(gbuf, [jnp.full((16,), gi + 1, jnp.int32)])
            for c in range(D_MODEL // 16):
                sl = pl.ds(c * 16, 16)
                outbuf[i, sl] = (g0 * rowbuf[2 * i, sl] +
                                 g1 * rowbuf[2 * i + 1, sl] + biasbuf[i, sl])
            return 0

        lax.fori_loop(0, CTOK, tok_body, 0)
        wh[j] = pltpu.async_copy(
            outbuf, out_hbm.at[pl.ds(tbase + j * CTOK, CTOK)], osem)
    wh[nch - 2].wait()
    wh[nch - 1].wait()


def _combine(posflat, gates, bias_comb, ys):
    mesh = plsc.VectorSubcoreMesh(core_axis_name="c", subcore_axis_name="s")
    return pl.kernel(
        _combine_body,
        mesh=mesh,
        out_type=jax.ShapeDtypeStruct((TOKENS, D_MODEL), jnp.float32),
        compiler_params=pltpu.CompilerParams(needs_layout_passes=False),
        scratch_types=[
            pltpu.VMEM((TOP_K * TOK_PER_W,), jnp.int32),
            pltpu.VMEM((TOP_K * TOK_PER_W,), jnp.float32),
            pltpu.VMEM((2 * CTOK, D_MODEL), jnp.float32),
            pltpu.VMEM((2 * CTOK, D_MODEL), jnp.float32),
            pltpu.VMEM((CTOK, D_MODEL), jnp.float32),
            pltpu.VMEM((CTOK, D_MODEL), jnp.float32),
            pltpu.VMEM((CTOK, D_MODEL), jnp.float32),
            pltpu.VMEM((CTOK, D_MODEL), jnp.float32),
            pltpu.SemaphoreType.DMA,
            pltpu.SemaphoreType.DMA,
            pltpu.SemaphoreType.DMA,
        ],
    )(posflat, gates, bias_comb, ys)


def kernel(inputs, Wg, bg, We, be):
    xb = inputs.astype(jnp.bfloat16)
    xb32 = lax.bitcast_convert_type(
        xb.reshape(TOKENS, D32, 2), jnp.int32)

    probs, posflat2, bexp_row, hcomb = _router(inputs, Wg, bg)
    bias_comb = _bias_comb(hcomb, be)
    posflat = posflat2.reshape(TOKENS * TOP_K)
    gates = probs.reshape(TOKENS * TOP_K)
    block_expert = bexp_row[0, :NBLK]

    xs32 = _dispatch(posflat, xb32)
    xs_bf16 = lax.bitcast_convert_type(xs32, jnp.bfloat16).reshape(
        PADDED, D_MODEL)
    ys = _grouped_matmul(block_expert, xs_bf16, We.astype(jnp.bfloat16))
    out = _combine(posflat, gates, bias_comb, ys)
    return (out, probs)


# speedup vs baseline: 5.5725x; 5.3744x over previous
import functools

import jax
import jax.numpy as jnp
from jax.experimental import pallas as pl
from jax.experimental.pallas import tpu as pltpu

TOP_K = 2
NUM_EXPERTS = 8
D_MODEL = 1024
TOKENS = 4096
TBLK = 512


def _moe_block(xb_ref, wg_ref, bg_ref, we_ref, be_ref, out_ref, probs_ref):
    xb = xb_ref[...]
    scores = jnp.dot(xb, wg_ref[...], preferred_element_type=jnp.float32)
    scores = scores + bg_ref[...]
    idx = jax.lax.broadcasted_iota(jnp.int32, scores.shape, 1)
    m1 = jnp.max(scores, axis=1, keepdims=True)
    i1 = jnp.min(jnp.where(scores == m1, idx, NUM_EXPERTS), axis=1,
                 keepdims=True)
    masked = jnp.where(idx == i1, -jnp.inf, scores)
    m2 = jnp.max(masked, axis=1, keepdims=True)
    i2 = jnp.min(jnp.where(masked == m2, idx, NUM_EXPERTS), axis=1,
                 keepdims=True)
    e2 = jnp.exp(m2 - m1)
    denom = 1.0 + e2
    p0 = 1.0 / denom
    p1 = e2 / denom
    probs_ref[...] = jnp.concatenate([p0, p1], axis=1)
    gate = jnp.where(idx == i1, p0, 0.0) + jnp.where(idx == i2, p1, 0.0)

    acc0 = jnp.dot(gate, be_ref[...], preferred_element_type=jnp.float32)
    acc1 = jnp.zeros((TBLK, D_MODEL), jnp.float32)
    accs = [acc0, acc1]
    for e in range(NUM_EXPERTS):
        y = jnp.dot(xb, we_ref[e], preferred_element_type=jnp.float32)
        accs[e % 2] = accs[e % 2] + gate[:, e:e + 1] * y
    out_ref[...] = accs[0] + accs[1]


def kernel(inputs, Wg, bg, We, be):
    n_tb = TOKENS // TBLK
    out, probs = pl.pallas_call(
        _moe_block,
        grid=(n_tb,),
        in_specs=[
            pl.BlockSpec((TBLK, D_MODEL), lambda t: (t, 0)),
            pl.BlockSpec((D_MODEL, NUM_EXPERTS), lambda t: (0, 0)),
            pl.BlockSpec((1, NUM_EXPERTS), lambda t: (0, 0)),
            pl.BlockSpec((NUM_EXPERTS, D_MODEL, D_MODEL), lambda t: (0, 0, 0)),
            pl.BlockSpec((NUM_EXPERTS, D_MODEL), lambda t: (0, 0)),
        ],
        out_specs=[
            pl.BlockSpec((TBLK, D_MODEL), lambda t: (t, 0)),
            pl.BlockSpec((TBLK, TOP_K), lambda t: (t, 0)),
        ],
        out_shape=[
            jax.ShapeDtypeStruct((TOKENS, D_MODEL), jnp.float32),
            jax.ShapeDtypeStruct((TOKENS, TOP_K), jnp.float32),
        ],
    )(inputs.astype(jnp.bfloat16), Wg, bg.reshape(1, NUM_EXPERTS),
      We.astype(jnp.bfloat16), be)
    return (out, probs)
